# baseline (speedup 1.0000x reference)
import jax
import jax.numpy as jnp
from jax.experimental import pallas as pl
from jax.experimental.pallas import tpu as pltpu

_N = 512
_ALPHA = 0.02
_BETA = 0.1
_GAMMA = 0.02
_W1 = 0.5
_W2 = 0.5
_SHIFT = 0.0


def _pairwise_body(f_ref, ft_ref, aux_ref, auxt_ref, out_ref):
    n = _N
    cf = f_ref.shape[1]

    ft = ft_ref[0]
    auxt = auxt_ref[0]

    lane_c = jax.lax.broadcasted_iota(jnp.int32, (1, cf), 1)
    lane_a = jax.lax.broadcasted_iota(jnp.int32, (1, 8), 1)

    def aux_col(c):
        onehot = (lane_a == c).astype(jnp.float32)
        return jnp.sum(auxt * onehot, axis=1, keepdims=True)

    def body(c, acc):
        row = f_ref[0, pl.ds(c, 1), :]
        onehot = (lane_c == c).astype(jnp.float32)
        col = jnp.sum(ft * onehot, axis=1, keepdims=True)
        d = col - row
        ad = jnp.abs(d)
        sl1 = jnp.where(ad < 1.0, 0.5 * d * d, ad - 0.5)
        return acc + sl1

    acc = jax.lax.fori_loop(0, cf, body, jnp.zeros((n, n), jnp.float32))
    feat_mean = acc * (1.0 / cf)

    def sqdiff(c):
        row = aux_ref[0, c:c + 1, :]
        col = aux_col(c)
        d = col - row
        return d * d

    gd = sqdiff(0) + sqdiff(1) + sqdiff(2)
    cd = sqdiff(3) + sqdiff(4)

    e1 = -(cd * (1.0 / (2.0 * _ALPHA)) + gd * (1.0 / (2.0 * _BETA)))
    e2 = -(cd * (1.0 / (2.0 * _GAMMA)))
    sim = (_W1 * jnp.exp(e1) + _W2 * jnp.exp(e2) - _SHIFT) * (1.0 / (_W1 + _W2))

    vprod = aux_col(5) * aux_ref[0, 5:6, :]
    sprod = aux_col(6) * aux_ref[0, 6:7, :]
    unc = jnp.sqrt(jnp.maximum(sprod, 1e-8))

    t = vprod * feat_mean * sim
    s_loss = jnp.sum(unc * t)
    s_raw = jnp.sum(t)
    s_v = jnp.sum(vprod)

    out_ref[0] = jnp.concatenate(
        [jnp.full((1, 128), s_loss, jnp.float32),
         jnp.full((1, 128), s_raw, jnp.float32),
         jnp.full((1, 128), s_v, jnp.float32)], axis=1)


def _pairwise_call(sel_feats, ft, aux, auxt):
    b = sel_feats.shape[0]
    grid = (b,)
    return pl.pallas_call(
        _pairwise_body,
        grid=grid,
        in_specs=[
            pl.BlockSpec((1, sel_feats.shape[1], _N), lambda i: (i, 0, 0)),
            pl.BlockSpec((1, _N, ft.shape[2]), lambda i: (i, 0, 0)),
            pl.BlockSpec((1, 8, _N), lambda i: (i, 0, 0)),
            pl.BlockSpec((1, _N, 8), lambda i: (i, 0, 0)),
        ],
        out_specs=pl.BlockSpec((1, 1, 384), lambda i: (i, 0, 0)),
        out_shape=jax.ShapeDtypeStruct((b, 1, 384), jnp.float32),
    )(sel_feats, ft, aux, auxt)


def kernel(guidance, features, valid_mask, loss_scales):
    b, cg, h, w = guidance.shape
    cf = features.shape[1]
    n = _N

    flat_valid = valid_mask.reshape(b, -1)
    u = jax.random.uniform(jax.random.key(42), flat_valid.shape, dtype=jnp.float32)
    g = -jnp.log(-jnp.log(u + 1e-20) + 1e-20)
    scores = jnp.log(flat_valid + 1e-8) + g
    _, offs = jax.lax.top_k(scores, n)

    feats_flat = features.reshape(b, cf, -1)
    guid_flat = guidance.reshape(b, cg, -1)
    sel_feats = jnp.take_along_axis(feats_flat, offs[:, None, :], axis=2)
    sel_guid = jnp.take_along_axis(guid_flat, offs[:, None, :], axis=2)
    sel_valid = jnp.take_along_axis(flat_valid, offs, axis=1)
    sel_scales = jnp.take_along_axis(loss_scales.reshape(b, -1), offs, axis=1)

    y = (offs // w).astype(jnp.float32) * (1.0 / h)
    x = (offs % w).astype(jnp.float32) * (1.0 / w)
    aux = jnp.concatenate(
        [sel_guid, y[:, None, :], x[:, None, :], sel_valid[:, None, :],
         sel_scales[:, None, :], jnp.zeros((b, 1, n), jnp.float32)], axis=1)

    ft = sel_feats.transpose(0, 2, 1)
    auxt = aux.transpose(0, 2, 1)

    part = _pairwise_call(sel_feats, ft, aux, auxt)
    s_loss = part[:, 0, 0].sum()
    s_raw = part[:, 0, 128].sum()
    s_v = part[:, 0, 256].sum()

    div = jnp.maximum(s_v, (b * n * n) / 2.0)
    loss = s_loss / div
    raw_loss = jax.lax.stop_gradient(s_raw) / div
    return (loss, raw_loss)
